# initial kernel scaffold (unmeasured)
import jax
import jax.numpy as jnp
from jax import lax
from jax.experimental import pallas as pl
from jax.experimental.pallas import tpu as pltpu

N_DEV = 4
B = 2
S_LOC = 512
S = 2048
D = 1024
H_LOC = 8
DH = 128
SCALE = 0.08838834764831843


def _neighbor_barrier(left, right):
    barrier = pltpu.get_barrier_semaphore()
    for nbr in (left, right):
        pl.semaphore_signal(
            barrier, inc=1, device_id=(nbr,),
            device_id_type=pl.DeviceIdType.MESH,
        )
    pl.semaphore_wait(barrier, 2)


def _ag_body(x_ref, out_ref, send_sems, recv_sems):
    my = lax.axis_index("i")
    left = lax.rem(my + N_DEV - 1, N_DEV)
    right = lax.rem(my + 1, N_DEV)

    _neighbor_barrier(left, right)

    out_ref[:, pl.ds(my * S_LOC, S_LOC), :] = x_ref[...]

    for h in range(N_DEV - 1):
        origin = lax.rem(my + N_DEV - h, N_DEV)
        sl = pl.ds(origin * S_LOC, S_LOC)
        rdma = pltpu.make_async_remote_copy(
            src_ref=out_ref.at[:, sl, :],
            dst_ref=out_ref.at[:, sl, :],
            send_sem=send_sems.at[h],
            recv_sem=recv_sems.at[h],
            device_id=(right,),
            device_id_type=pl.DeviceIdType.MESH,
        )
        rdma.start()
        rdma.wait()


def _all_gather(x):
    return pl.pallas_call(
        _ag_body,
        out_shape=jax.ShapeDtypeStruct((B, S, D), jnp.float32),
        in_specs=[pl.BlockSpec(memory_space=pltpu.VMEM)],
        out_specs=pl.BlockSpec(memory_space=pltpu.VMEM),
        scratch_shapes=[
            pltpu.SemaphoreType.DMA((N_DEV - 1,)),
            pltpu.SemaphoreType.DMA((N_DEV - 1,)),
        ],
        compiler_params=pltpu.CompilerParams(collective_id=0),
    )(x)


def _rope_tables():
    pos = lax.broadcasted_iota(jnp.int32, (S, DH), 0).astype(jnp.float32)
    d = lax.broadcasted_iota(jnp.int32, (S, DH), 1)
    half = (d // 2).astype(jnp.float32)
    inv = jnp.exp(-jnp.log(10000.0) * (half * (2.0 / DH)))
    ang = pos * inv
    return jnp.cos(ang), jnp.sin(ang), (d % 2) == 0


def _rope(t, cos, sin, even):
    l = jnp.concatenate([t[:, 1:], t[:, :1]], axis=1)
    r = jnp.concatenate([t[:, -1:], t[:, :-1]], axis=1)
    tr = jnp.where(even, -l, r)
    return t * cos + tr * sin


def _attn_body(x_ref, wq_ref, wk_ref, wv_ref, wo_ref, out_ref):
    h = pl.program_id(1)
    cos, sin, even = _rope_tables()
    xb = x_ref[0]
    q = _rope(jnp.dot(xb, wq_ref[...]), cos, sin, even)
    k = _rope(jnp.dot(xb, wk_ref[...]), cos, sin, even)
    v = jnp.dot(xb, wv_ref[...])
    for qb in range(S // S_LOC):
        qs = q[qb * S_LOC:(qb + 1) * S_LOC]
        s = lax.dot_general(qs, k, (((1,), (1,)), ((), ()))) * SCALE
        m = jnp.max(s, axis=1, keepdims=True)
        p = jnp.exp(s - m)
        p = p / jnp.sum(p, axis=1, keepdims=True)
        ctx = jnp.dot(p, v)
        contrib = jnp.dot(ctx, wo_ref[...])
        sl = pl.ds(qb * S_LOC, S_LOC)

        @pl.when(h == 0)
        def _():
            out_ref[0, sl, :] = contrib

        @pl.when(h != 0)
        def _():
            out_ref[0, sl, :] = out_ref[0, sl, :] + contrib


def _attention_partial(x_full, wq, wk, wv, wo):
    return pl.pallas_call(
        _attn_body,
        grid=(B, H_LOC),
        out_shape=jax.ShapeDtypeStruct((B, S, D), jnp.float32),
        in_specs=[
            pl.BlockSpec((1, S, D), lambda b, h: (b, 0, 0)),
            pl.BlockSpec((D, DH), lambda b, h: (0, h)),
            pl.BlockSpec((D, DH), lambda b, h: (0, h)),
            pl.BlockSpec((D, DH), lambda b, h: (0, h)),
            pl.BlockSpec((DH, D), lambda b, h: (h, 0)),
        ],
        out_specs=pl.BlockSpec((1, S, D), lambda b, h: (b, 0, 0)),
    )(x_full, wq, wk, wv, wo)


def _rs_body(p_ref, out_ref, rsbuf, accbuf, send_sems, recv_sems):
    my = lax.axis_index("i")
    left = lax.rem(my + N_DEV - 1, N_DEV)
    right = lax.rem(my + 1, N_DEV)

    _neighbor_barrier(left, right)

    for s in range(N_DEV - 1):
        c_send = lax.rem(my + 2 * N_DEV - 1 - s, N_DEV)
        if s == 0:
            src = p_ref.at[:, pl.ds(c_send * S_LOC, S_LOC), :]
        else:
            src = accbuf
        rdma = pltpu.make_async_remote_copy(
            src_ref=src,
            dst_ref=rsbuf.at[s],
            send_sem=send_sems.at[s],
            recv_sem=recv_sems.at[s],
            device_id=(right,),
            device_id_type=pl.DeviceIdType.MESH,
        )
        rdma.start()
        rdma.wait()
        c_recv = lax.rem(my + 2 * N_DEV - 2 - s, N_DEV)
        csl = pl.ds(c_recv * S_LOC, S_LOC)
        if s < N_DEV - 2:
            accbuf[...] = rsbuf[s] + p_ref[:, csl, :]
        else:
            out_ref[...] = rsbuf[s] + p_ref[:, csl, :]


def _reduce_scatter(partial):
    return pl.pallas_call(
        _rs_body,
        out_shape=jax.ShapeDtypeStruct((B, S_LOC, D), jnp.float32),
        in_specs=[pl.BlockSpec(memory_space=pltpu.VMEM)],
        out_specs=pl.BlockSpec(memory_space=pltpu.VMEM),
        scratch_shapes=[
            pltpu.VMEM((N_DEV - 1, B, S_LOC, D), jnp.float32),
            pltpu.VMEM((B, S_LOC, D), jnp.float32),
            pltpu.SemaphoreType.DMA((N_DEV - 1,)),
            pltpu.SemaphoreType.DMA((N_DEV - 1,)),
        ],
        compiler_params=pltpu.CompilerParams(collective_id=1),
    )(partial)


def kernel(x, Wq, Wk, Wv, Wo):
    x_full = _all_gather(x)
    partial = _attention_partial(x_full, Wq, Wk, Wv, Wo)
    return _reduce_scatter(partial)


# baseline (device time: 624780 ns/iter reference)
import jax
import jax.numpy as jnp
from jax import lax
from jax.experimental import pallas as pl
from jax.experimental.pallas import tpu as pltpu

N_DEV = 4
B = 2
S_LOC = 512
S = 2048
D = 1024
H_LOC = 8
DH = 128
SCALE = 0.08838834764831843


def _neighbor_barrier(left, right):
    barrier = pltpu.get_barrier_semaphore()
    for nbr in (left, right):
        pl.semaphore_signal(
            barrier, inc=1, device_id=(nbr,),
            device_id_type=pl.DeviceIdType.MESH,
        )
    pl.semaphore_wait(barrier, 2)


def _ag_body(x_ref, out_ref, send_sems, recv_sems):
    my = lax.axis_index("i")
    left = lax.rem(my + N_DEV - 1, N_DEV)
    right = lax.rem(my + 1, N_DEV)

    _neighbor_barrier(left, right)

    out_ref[:, pl.ds(my * S_LOC, S_LOC), :] = x_ref[...]

    for h in range(N_DEV - 1):
        origin = lax.rem(my + N_DEV - h, N_DEV)
        sl = pl.ds(origin * S_LOC, S_LOC)
        rdma = pltpu.make_async_remote_copy(
            src_ref=out_ref.at[:, sl, :],
            dst_ref=out_ref.at[:, sl, :],
            send_sem=send_sems.at[h],
            recv_sem=recv_sems.at[h],
            device_id=(right,),
            device_id_type=pl.DeviceIdType.MESH,
        )
        rdma.start()
        rdma.wait()


def _all_gather(x):
    return pl.pallas_call(
        _ag_body,
        out_shape=jax.ShapeDtypeStruct((B, S, D), jnp.float32),
        in_specs=[pl.BlockSpec(memory_space=pltpu.VMEM)],
        out_specs=pl.BlockSpec(memory_space=pltpu.VMEM),
        scratch_shapes=[
            pltpu.SemaphoreType.DMA((N_DEV - 1,)),
            pltpu.SemaphoreType.DMA((N_DEV - 1,)),
        ],
        compiler_params=pltpu.CompilerParams(collective_id=0),
    )(x)


def _rope_tables():
    pos = lax.broadcasted_iota(jnp.int32, (S, DH), 0).astype(jnp.float32)
    d = lax.broadcasted_iota(jnp.int32, (S, DH), 1)
    half = (d // 2).astype(jnp.float32)
    inv = jnp.exp(-jnp.log(10000.0) * (half * (2.0 / DH)))
    ang = pos * inv
    return jnp.cos(ang), jnp.sin(ang), (d % 2) == 0


def _rope(t, cos, sin, even):
    l = jnp.concatenate([t[:, 1:], t[:, :1]], axis=1)
    r = jnp.concatenate([t[:, -1:], t[:, :-1]], axis=1)
    tr = jnp.where(even, -l, r)
    return t * cos + tr * sin


def _attn_body(x_ref, wq_ref, wk_ref, wv_ref, wo_ref, out_ref):
    h = pl.program_id(1)
    cos, sin, even = _rope_tables()
    xb = x_ref[0]
    q = _rope(jnp.dot(xb, wq_ref[...]), cos, sin, even)
    k = _rope(jnp.dot(xb, wk_ref[...]), cos, sin, even)
    v = jnp.dot(xb, wv_ref[...])
    for qb in range(S // S_LOC):
        qs = q[qb * S_LOC:(qb + 1) * S_LOC]
        s = lax.dot_general(qs, k, (((1,), (1,)), ((), ()))) * SCALE
        m = jnp.max(s, axis=1, keepdims=True)
        p = jnp.exp(s - m)
        p = p / jnp.sum(p, axis=1, keepdims=True)
        ctx = jnp.dot(p, v)
        contrib = jnp.dot(ctx, wo_ref[...])
        sl = pl.ds(qb * S_LOC, S_LOC)

        @pl.when(h == 0)
        def _():
            out_ref[0, sl, :] = contrib

        @pl.when(h != 0)
        def _():
            out_ref[0, sl, :] = out_ref[0, sl, :] + contrib


def _attention_partial(x_full, wq, wk, wv, wo):
    return pl.pallas_call(
        _attn_body,
        grid=(B, H_LOC),
        out_shape=jax.ShapeDtypeStruct((B, S, D), jnp.float32),
        in_specs=[
            pl.BlockSpec((1, S, D), lambda b, h: (b, 0, 0)),
            pl.BlockSpec((D, DH), lambda b, h: (0, h)),
            pl.BlockSpec((D, DH), lambda b, h: (0, h)),
            pl.BlockSpec((D, DH), lambda b, h: (0, h)),
            pl.BlockSpec((DH, D), lambda b, h: (h, 0)),
        ],
        out_specs=pl.BlockSpec((1, S, D), lambda b, h: (b, 0, 0)),
        compiler_params=pltpu.CompilerParams(
            vmem_limit_bytes=64 * 1024 * 1024,
        ),
    )(x_full, wq, wk, wv, wo)


def _rs_body(p_ref, out_ref, rsbuf, accbuf, send_sems, recv_sems):
    my = lax.axis_index("i")
    left = lax.rem(my + N_DEV - 1, N_DEV)
    right = lax.rem(my + 1, N_DEV)

    _neighbor_barrier(left, right)

    for s in range(N_DEV - 1):
        c_send = lax.rem(my + 2 * N_DEV - 1 - s, N_DEV)
        if s == 0:
            src = p_ref.at[:, pl.ds(c_send * S_LOC, S_LOC), :]
        else:
            src = accbuf
        rdma = pltpu.make_async_remote_copy(
            src_ref=src,
            dst_ref=rsbuf.at[s],
            send_sem=send_sems.at[s],
            recv_sem=recv_sems.at[s],
            device_id=(right,),
            device_id_type=pl.DeviceIdType.MESH,
        )
        rdma.start()
        rdma.wait()
        c_recv = lax.rem(my + 2 * N_DEV - 2 - s, N_DEV)
        csl = pl.ds(c_recv * S_LOC, S_LOC)
        if s < N_DEV - 2:
            accbuf[...] = rsbuf[s] + p_ref[:, csl, :]
        else:
            out_ref[...] = rsbuf[s] + p_ref[:, csl, :]


def _reduce_scatter(partial):
    return pl.pallas_call(
        _rs_body,
        out_shape=jax.ShapeDtypeStruct((B, S_LOC, D), jnp.float32),
        in_specs=[pl.BlockSpec(memory_space=pltpu.VMEM)],
        out_specs=pl.BlockSpec(memory_space=pltpu.VMEM),
        scratch_shapes=[
            pltpu.VMEM((N_DEV - 1, B, S_LOC, D), jnp.float32),
            pltpu.VMEM((B, S_LOC, D), jnp.float32),
            pltpu.SemaphoreType.DMA((N_DEV - 1,)),
            pltpu.SemaphoreType.DMA((N_DEV - 1,)),
        ],
        compiler_params=pltpu.CompilerParams(collective_id=1),
    )(partial)


def kernel(x, Wq, Wk, Wv, Wo):
    x_full = _all_gather(x)
    partial = _attention_partial(x_full, Wq, Wk, Wv, Wo)
    return _reduce_scatter(partial)


# device time: 493132 ns/iter; 1.2670x vs baseline; 1.2670x over previous
import jax
import jax.numpy as jnp
from jax import lax
from jax.experimental import pallas as pl
from jax.experimental.pallas import tpu as pltpu

N_DEV = 4
B = 2
S_LOC = 512
S = 2048
D = 1024
H_LOC = 8
DH = 128
SCALE = 0.08838834764831843


def _neighbor_barrier(left, right):
    barrier = pltpu.get_barrier_semaphore()
    for nbr in (left, right):
        pl.semaphore_signal(
            barrier, inc=1, device_id=(nbr,),
            device_id_type=pl.DeviceIdType.MESH,
        )
    pl.semaphore_wait(barrier, 2)


def _ag_body(x_ref, out_ref, send_sems, recv_sems):
    my = lax.axis_index("i")
    left = lax.rem(my + N_DEV - 1, N_DEV)
    right = lax.rem(my + 1, N_DEV)

    _neighbor_barrier(left, right)

    out_ref[:, pl.ds(my * S_LOC, S_LOC), :] = x_ref[...].astype(jnp.bfloat16)

    for h in range(N_DEV - 1):
        origin = lax.rem(my + N_DEV - h, N_DEV)
        sl = pl.ds(origin * S_LOC, S_LOC)
        rdma = pltpu.make_async_remote_copy(
            src_ref=out_ref.at[:, sl, :],
            dst_ref=out_ref.at[:, sl, :],
            send_sem=send_sems.at[h],
            recv_sem=recv_sems.at[h],
            device_id=(right,),
            device_id_type=pl.DeviceIdType.MESH,
        )
        rdma.start()
        rdma.wait()


def _all_gather(x):
    return pl.pallas_call(
        _ag_body,
        out_shape=jax.ShapeDtypeStruct((B, S, D), jnp.bfloat16),
        in_specs=[pl.BlockSpec(memory_space=pltpu.VMEM)],
        out_specs=pl.BlockSpec(memory_space=pltpu.VMEM),
        scratch_shapes=[
            pltpu.SemaphoreType.DMA((N_DEV - 1,)),
            pltpu.SemaphoreType.DMA((N_DEV - 1,)),
        ],
        compiler_params=pltpu.CompilerParams(collective_id=0),
    )(x)


def _rope_tables():
    pos = lax.broadcasted_iota(jnp.int32, (S, DH), 0).astype(jnp.float32)
    d = lax.broadcasted_iota(jnp.int32, (S, DH), 1)
    half = (d // 2).astype(jnp.float32)
    inv = jnp.exp(-jnp.log(10000.0) * (half * (2.0 / DH)))
    ang = pos * inv
    return jnp.cos(ang), jnp.sin(ang), (d % 2) == 0


def _rope(t, cos, sin, even):
    l = jnp.concatenate([t[:, 1:], t[:, :1]], axis=1)
    r = jnp.concatenate([t[:, -1:], t[:, :-1]], axis=1)
    tr = jnp.where(even, -l, r)
    return t * cos + tr * sin


def _attn_body(x_ref, wq_ref, wk_ref, wv_ref, wo_ref, out_ref):
    h = pl.program_id(1)
    f32 = jnp.float32
    bf16 = jnp.bfloat16
    cos, sin, even = _rope_tables()
    xb = x_ref[0]
    wq = wq_ref[...].astype(bf16)
    wk = wk_ref[...].astype(bf16)
    wv = wv_ref[...].astype(bf16)
    wo = wo_ref[...].astype(bf16)
    q = _rope(jnp.dot(xb, wq, preferred_element_type=f32), cos, sin, even)
    k = _rope(jnp.dot(xb, wk, preferred_element_type=f32), cos, sin, even)
    qb16 = q.astype(bf16)
    kb16 = k.astype(bf16)
    v = jnp.dot(xb, wv, preferred_element_type=f32).astype(bf16)
    for qb in range(S // S_LOC):
        qs = qb16[qb * S_LOC:(qb + 1) * S_LOC]
        s = lax.dot_general(
            qs, kb16, (((1,), (1,)), ((), ())),
            preferred_element_type=f32,
        ) * SCALE
        m = jnp.max(s, axis=1, keepdims=True)
        p = jnp.exp(s - m)
        p = (p / jnp.sum(p, axis=1, keepdims=True)).astype(bf16)
        ctx = jnp.dot(p, v, preferred_element_type=f32).astype(bf16)
        contrib = jnp.dot(ctx, wo, preferred_element_type=f32)
        sl = pl.ds(qb * S_LOC, S_LOC)

        @pl.when(h == 0)
        def _():
            out_ref[0, sl, :] = contrib

        @pl.when(h != 0)
        def _():
            out_ref[0, sl, :] = out_ref[0, sl, :] + contrib


def _attention_partial(x_full, wq, wk, wv, wo):
    return pl.pallas_call(
        _attn_body,
        grid=(B, H_LOC),
        out_shape=jax.ShapeDtypeStruct((B, S, D), jnp.float32),
        in_specs=[
            pl.BlockSpec((1, S, D), lambda b, h: (b, 0, 0)),
            pl.BlockSpec((D, DH), lambda b, h: (0, h)),
            pl.BlockSpec((D, DH), lambda b, h: (0, h)),
            pl.BlockSpec((D, DH), lambda b, h: (0, h)),
            pl.BlockSpec((DH, D), lambda b, h: (h, 0)),
        ],
        out_specs=pl.BlockSpec((1, S, D), lambda b, h: (b, 0, 0)),
        compiler_params=pltpu.CompilerParams(
            vmem_limit_bytes=64 * 1024 * 1024,
        ),
    )(x_full, wq, wk, wv, wo)


def _rs_body(p_ref, out_ref, rsbuf, sendbuf, send_sems, recv_sems):
    my = lax.axis_index("i")
    left = lax.rem(my + N_DEV - 1, N_DEV)
    right = lax.rem(my + 1, N_DEV)

    _neighbor_barrier(left, right)

    c0 = lax.rem(my + N_DEV - 1, N_DEV)
    sendbuf[...] = p_ref[:, pl.ds(c0 * S_LOC, S_LOC), :].astype(jnp.bfloat16)
    for s in range(N_DEV - 1):
        rdma = pltpu.make_async_remote_copy(
            src_ref=sendbuf,
            dst_ref=rsbuf.at[s],
            send_sem=send_sems.at[s],
            recv_sem=recv_sems.at[s],
            device_id=(right,),
            device_id_type=pl.DeviceIdType.MESH,
        )
        rdma.start()
        rdma.wait()
        c_recv = lax.rem(my + 2 * N_DEV - 2 - s, N_DEV)
        csl = pl.ds(c_recv * S_LOC, S_LOC)
        acc = rsbuf[s].astype(jnp.float32) + p_ref[:, csl, :]
        if s < N_DEV - 2:
            sendbuf[...] = acc.astype(jnp.bfloat16)
        else:
            out_ref[...] = acc


def _reduce_scatter(partial):
    return pl.pallas_call(
        _rs_body,
        out_shape=jax.ShapeDtypeStruct((B, S_LOC, D), jnp.float32),
        in_specs=[pl.BlockSpec(memory_space=pltpu.VMEM)],
        out_specs=pl.BlockSpec(memory_space=pltpu.VMEM),
        scratch_shapes=[
            pltpu.VMEM((N_DEV - 1, B, S_LOC, D), jnp.bfloat16),
            pltpu.VMEM((B, S_LOC, D), jnp.bfloat16),
            pltpu.SemaphoreType.DMA((N_DEV - 1,)),
            pltpu.SemaphoreType.DMA((N_DEV - 1,)),
        ],
        compiler_params=pltpu.CompilerParams(collective_id=1),
    )(partial)


def kernel(x, Wq, Wk, Wv, Wo):
    x_full = _all_gather(x)
    partial = _attention_partial(x_full, Wq, Wk, Wv, Wo)
    return _reduce_scatter(partial)


# device time: 430975 ns/iter; 1.4497x vs baseline; 1.1442x over previous
import jax
import jax.numpy as jnp
from jax import lax
from jax.experimental import pallas as pl
from jax.experimental.pallas import tpu as pltpu

N_DEV = 4
B = 2
S_LOC = 512
S = 2048
D = 1024
H_LOC = 8
DH = 128
SCALE = 0.08838834764831843


def _neighbor_barrier(left, right):
    barrier = pltpu.get_barrier_semaphore()
    for nbr in (left, right):
        pl.semaphore_signal(
            barrier, inc=1, device_id=(nbr,),
            device_id_type=pl.DeviceIdType.MESH,
        )
    pl.semaphore_wait(barrier, 2)


def _ag_body(x_ref, out_ref, send_sems, recv_sems):
    my = lax.axis_index("i")
    left = lax.rem(my + N_DEV - 1, N_DEV)
    right = lax.rem(my + 1, N_DEV)

    _neighbor_barrier(left, right)

    out_ref[:, pl.ds(my * S_LOC, S_LOC), :] = x_ref[...].astype(jnp.bfloat16)

    for h in range(N_DEV - 1):
        origin = lax.rem(my + N_DEV - h, N_DEV)
        sl = pl.ds(origin * S_LOC, S_LOC)
        rdma = pltpu.make_async_remote_copy(
            src_ref=out_ref.at[:, sl, :],
            dst_ref=out_ref.at[:, sl, :],
            send_sem=send_sems.at[h],
            recv_sem=recv_sems.at[h],
            device_id=(right,),
            device_id_type=pl.DeviceIdType.MESH,
        )
        rdma.start()
        rdma.wait()


def _all_gather(x):
    return pl.pallas_call(
        _ag_body,
        out_shape=jax.ShapeDtypeStruct((B, S, D), jnp.bfloat16),
        in_specs=[pl.BlockSpec(memory_space=pltpu.VMEM)],
        out_specs=pl.BlockSpec(memory_space=pltpu.VMEM),
        scratch_shapes=[
            pltpu.SemaphoreType.DMA((N_DEV - 1,)),
            pltpu.SemaphoreType.DMA((N_DEV - 1,)),
        ],
        compiler_params=pltpu.CompilerParams(collective_id=0),
    )(x)


def _rope_tables():
    pos = lax.broadcasted_iota(jnp.int32, (S, DH), 0).astype(jnp.float32)
    d = lax.broadcasted_iota(jnp.int32, (S, DH), 1)
    half = (d // 2).astype(jnp.float32)
    inv = jnp.exp(-jnp.log(10000.0) * (half * (2.0 / DH)))
    ang = pos * inv
    return jnp.cos(ang), jnp.sin(ang), (d % 2) == 0


def _rope(t, cos, sin, even):
    l = jnp.concatenate([t[:, 1:], t[:, :1]], axis=1)
    r = jnp.concatenate([t[:, -1:], t[:, :-1]], axis=1)
    tr = jnp.where(even, -l, r)
    return t * cos + tr * sin


def _attn_body(x_ref, wq_ref, wk_ref, wv_ref, wo_ref, out_ref):
    h = pl.program_id(1)
    f32 = jnp.float32
    bf16 = jnp.bfloat16
    cos, sin, even = _rope_tables()
    xb = x_ref[0]
    wq = wq_ref[...].astype(bf16)
    wk = wk_ref[...].astype(bf16)
    wv = wv_ref[...].astype(bf16)
    wo = wo_ref[...].astype(bf16)
    q = _rope(jnp.dot(xb, wq, preferred_element_type=f32), cos, sin, even)
    k = _rope(jnp.dot(xb, wk, preferred_element_type=f32), cos, sin, even)
    qb16 = (q * SCALE).astype(bf16)
    kb16 = k.astype(bf16)
    v = jnp.dot(xb, wv, preferred_element_type=f32).astype(bf16)
    for qb in range(S // S_LOC):
        qs = qb16[qb * S_LOC:(qb + 1) * S_LOC]
        s = lax.dot_general(
            qs, kb16, (((1,), (1,)), ((), ())),
            preferred_element_type=f32,
        )
        p = jnp.exp(s)
        denom = jnp.sum(p, axis=1, keepdims=True)
        ctx_un = jnp.dot(p.astype(bf16), v, preferred_element_type=f32)
        ctx = (ctx_un / denom).astype(bf16)
        contrib = jnp.dot(ctx, wo, preferred_element_type=f32)
        sl = pl.ds(qb * S_LOC, S_LOC)

        @pl.when(h == 0)
        def _():
            out_ref[0, sl, :] = contrib

        @pl.when(h != 0)
        def _():
            out_ref[0, sl, :] = out_ref[0, sl, :] + contrib


def _attention_partial(x_full, wq, wk, wv, wo):
    return pl.pallas_call(
        _attn_body,
        grid=(B, H_LOC),
        out_shape=jax.ShapeDtypeStruct((B, S, D), jnp.float32),
        in_specs=[
            pl.BlockSpec((1, S, D), lambda b, h: (b, 0, 0)),
            pl.BlockSpec((D, DH), lambda b, h: (0, h)),
            pl.BlockSpec((D, DH), lambda b, h: (0, h)),
            pl.BlockSpec((D, DH), lambda b, h: (0, h)),
            pl.BlockSpec((DH, D), lambda b, h: (h, 0)),
        ],
        out_specs=pl.BlockSpec((1, S, D), lambda b, h: (b, 0, 0)),
        compiler_params=pltpu.CompilerParams(
            vmem_limit_bytes=64 * 1024 * 1024,
        ),
    )(x_full, wq, wk, wv, wo)


def _rs_body(p_ref, out_ref, rsbuf, sendbuf, send_sems, recv_sems):
    my = lax.axis_index("i")
    left = lax.rem(my + N_DEV - 1, N_DEV)
    right = lax.rem(my + 1, N_DEV)

    _neighbor_barrier(left, right)

    c0 = lax.rem(my + N_DEV - 1, N_DEV)
    sendbuf[...] = p_ref[:, pl.ds(c0 * S_LOC, S_LOC), :].astype(jnp.bfloat16)
    for s in range(N_DEV - 1):
        rdma = pltpu.make_async_remote_copy(
            src_ref=sendbuf,
            dst_ref=rsbuf.at[s],
            send_sem=send_sems.at[s],
            recv_sem=recv_sems.at[s],
            device_id=(right,),
            device_id_type=pl.DeviceIdType.MESH,
        )
        rdma.start()
        rdma.wait()
        c_recv = lax.rem(my + 2 * N_DEV - 2 - s, N_DEV)
        csl = pl.ds(c_recv * S_LOC, S_LOC)
        acc = rsbuf[s].astype(jnp.float32) + p_ref[:, csl, :]
        if s < N_DEV - 2:
            sendbuf[...] = acc.astype(jnp.bfloat16)
        else:
            out_ref[...] = acc


def _reduce_scatter(partial):
    return pl.pallas_call(
        _rs_body,
        out_shape=jax.ShapeDtypeStruct((B, S_LOC, D), jnp.float32),
        in_specs=[pl.BlockSpec(memory_space=pltpu.VMEM)],
        out_specs=pl.BlockSpec(memory_space=pltpu.VMEM),
        scratch_shapes=[
            pltpu.VMEM((N_DEV - 1, B, S_LOC, D), jnp.bfloat16),
            pltpu.VMEM((B, S_LOC, D), jnp.bfloat16),
            pltpu.SemaphoreType.DMA((N_DEV - 1,)),
            pltpu.SemaphoreType.DMA((N_DEV - 1,)),
        ],
        compiler_params=pltpu.CompilerParams(collective_id=1),
    )(partial)


def kernel(x, Wq, Wk, Wv, Wo):
    x_full = _all_gather(x)
    partial = _attention_partial(x_full, Wq, Wk, Wv, Wo)
    return _reduce_scatter(partial)
